# baseline (speedup 1.0000x reference)
import jax
import jax.numpy as jnp
from jax.experimental import pallas as pl


_EPB = 4


def _moe_body(x_ref, idx_ref, w_ref, wg_ref, wu_ref, dn_ref, out_ref):
    g = pl.program_id(0)
    x = x_ref[...]
    acc = None
    for ee in range(_EPB):
        e = g * _EPB + ee
        gate = jax.lax.dot_general(
            x, wg_ref[ee, 0], (((1,), (1,)), ((), ())),
            preferred_element_type=jnp.float32)
        up = jax.lax.dot_general(
            x, wu_ref[ee, 0], (((1,), (1,)), ((), ())),
            preferred_element_type=jnp.float32)
        h = gate * jax.nn.sigmoid(gate) * up
        oe = jax.lax.dot_general(
            h, dn_ref[ee], (((1,), (1,)), ((), ())),
            preferred_element_type=jnp.float32)
        cw = jnp.sum(
            jnp.where(idx_ref[...] == e, w_ref[...], 0.0), axis=1)
        contrib = oe * cw[:, None]
        acc = contrib if acc is None else acc + contrib

    @pl.when(g == 0)
    def _init():
        out_ref[...] = acc

    @pl.when(g != 0)
    def _acc():
        out_ref[...] += acc


def kernel(hidden_states, top_k_index, top_k_weights, gate_up_proj, down_proj):
    T, H = hidden_states.shape
    E, I2, _ = gate_up_proj.shape
    I = down_proj.shape[-1]
    B = _EPB

    gu4 = gate_up_proj.reshape(E, 2, I, H)

    return pl.pallas_call(
        _moe_body,
        grid=(E // B,),
        in_specs=[
            pl.BlockSpec((T, H), lambda g: (0, 0)),
            pl.BlockSpec(top_k_index.shape, lambda g: (0, 0)),
            pl.BlockSpec(top_k_weights.shape, lambda g: (0, 0)),
            pl.BlockSpec((B, 1, I, H), lambda g: (g, 0, 0, 0)),
            pl.BlockSpec((B, 1, I, H), lambda g: (g, 1, 0, 0)),
            pl.BlockSpec((B, H, I), lambda g: (g, 0, 0)),
        ],
        out_specs=pl.BlockSpec((T, H), lambda g: (0, 0)),
        out_shape=jax.ShapeDtypeStruct((T, H), jnp.float32),
    )(hidden_states, top_k_index, top_k_weights, gu4, gu4, down_proj)


# speedup vs baseline: 1.0106x; 1.0106x over previous
import jax
import jax.numpy as jnp
from jax.experimental import pallas as pl


_EPB = 2


def _moe_body(x_ref, idx_ref, w_ref, wgu_ref, dn_ref, out_ref):
    g = pl.program_id(0)
    x = x_ref[...]
    acc = None
    for ee in range(_EPB):
        e = g * _EPB + ee
        gate = jax.lax.dot_general(
            x, wgu_ref[ee, 0], (((1,), (1,)), ((), ())),
            preferred_element_type=jnp.float32)
        up = jax.lax.dot_general(
            x, wgu_ref[ee, 1], (((1,), (1,)), ((), ())),
            preferred_element_type=jnp.float32)
        h = gate * jax.nn.sigmoid(gate) * up
        oe = jax.lax.dot_general(
            h, dn_ref[ee], (((1,), (1,)), ((), ())),
            preferred_element_type=jnp.float32)
        cw = jnp.sum(
            jnp.where(idx_ref[...] == e, w_ref[...], 0.0), axis=1)
        contrib = oe * cw[:, None]
        acc = contrib if acc is None else acc + contrib

    @pl.when(g == 0)
    def _init():
        out_ref[...] = acc

    @pl.when(g != 0)
    def _acc():
        out_ref[...] += acc


def kernel(hidden_states, top_k_index, top_k_weights, gate_up_proj, down_proj):
    T, H = hidden_states.shape
    E, I2, _ = gate_up_proj.shape
    I = down_proj.shape[-1]
    B = _EPB

    gu4 = gate_up_proj.reshape(E, 2, I, H)

    return pl.pallas_call(
        _moe_body,
        grid=(E // B,),
        in_specs=[
            pl.BlockSpec((T, H), lambda g: (0, 0)),
            pl.BlockSpec(top_k_index.shape, lambda g: (0, 0)),
            pl.BlockSpec(top_k_weights.shape, lambda g: (0, 0)),
            pl.BlockSpec((B, 2, I, H), lambda g: (g, 0, 0, 0)),
            pl.BlockSpec((B, H, I), lambda g: (g, 0, 0)),
        ],
        out_specs=pl.BlockSpec((T, H), lambda g: (0, 0)),
        out_shape=jax.ShapeDtypeStruct((T, H), jnp.float32),
    )(hidden_states, top_k_index, top_k_weights, gu4, down_proj)


# speedup vs baseline: 1.0241x; 1.0134x over previous
import jax
import jax.numpy as jnp
from jax.experimental import pallas as pl


_EPB = 2


def _moe_body(x_ref, idx_ref, w_ref, wg_ref, wu_ref, dn_ref, out_ref):
    g = pl.program_id(0)
    x = x_ref[...]
    acc = None
    for ee in range(_EPB):
        e = g * _EPB + ee
        gate = jax.lax.dot_general(
            x, wg_ref[ee, 0], (((1,), (1,)), ((), ())),
            preferred_element_type=jnp.float32)
        up = jax.lax.dot_general(
            x, wu_ref[ee, 0], (((1,), (1,)), ((), ())),
            preferred_element_type=jnp.float32)
        h = gate * jax.nn.sigmoid(gate) * up
        oe = jax.lax.dot_general(
            h, dn_ref[ee], (((1,), (1,)), ((), ())),
            preferred_element_type=jnp.float32)
        cw = jnp.sum(
            jnp.where(idx_ref[...] == e, w_ref[...], 0.0), axis=1)
        contrib = oe * cw[:, None]
        acc = contrib if acc is None else acc + contrib

    @pl.when(g == 0)
    def _init():
        out_ref[...] = acc

    @pl.when(g != 0)
    def _acc():
        out_ref[...] += acc


def kernel(hidden_states, top_k_index, top_k_weights, gate_up_proj, down_proj):
    T, H = hidden_states.shape
    E, I2, _ = gate_up_proj.shape
    I = down_proj.shape[-1]
    B = _EPB

    gu4 = gate_up_proj.reshape(E, 2, I, H)

    return pl.pallas_call(
        _moe_body,
        grid=(E // B,),
        in_specs=[
            pl.BlockSpec((T, H), lambda g: (0, 0)),
            pl.BlockSpec(top_k_index.shape, lambda g: (0, 0)),
            pl.BlockSpec(top_k_weights.shape, lambda g: (0, 0)),
            pl.BlockSpec((B, 1, I, H), lambda g: (g, 0, 0, 0)),
            pl.BlockSpec((B, 1, I, H), lambda g: (g, 1, 0, 0)),
            pl.BlockSpec((B, H, I), lambda g: (g, 0, 0)),
        ],
        out_specs=pl.BlockSpec((T, H), lambda g: (0, 0)),
        out_shape=jax.ShapeDtypeStruct((T, H), jnp.float32),
    )(hidden_states, top_k_index, top_k_weights, gu4, gu4, down_proj)
